# baseline (speedup 1.0000x reference)
import jax
import jax.numpy as jnp
from jax import lax
from jax.experimental import pallas as pl
from jax.experimental.pallas import tpu as pltpu

_VOCAB = 1000000
_DIM = 64
_CTX = 20
_HID = 128
_NT = 8192
_NT8 = _NT // 8
_NFULL = _VOCAB // _NT
_TAIL = _VOCAB - _NFULL * _NT
_NBLK = _NFULL + 1
_VPAD = _NBLK * _NT
_K = 6


def _embed_body(x_ref, emb_blk, w1_ref, b1_ref, h_ref, acc_ref):
    i = pl.program_id(0)

    @pl.when(i == 0)
    def _():
        acc_ref[...] = jnp.zeros_like(acc_ref)

    r = x_ref[i] % 8
    row = emb_blk[pl.ds(r, 1), :]
    acc_ref[...] += jnp.dot(row, w1_ref[0],
                            preferred_element_type=jnp.float32)

    @pl.when(i == _CTX - 1)
    def _():
        h_ref[...] = jnp.maximum(acc_ref[...] + b1_ref[...], 0.0)


def _stream_body(h_ref, w2_hbm, b2_ref, o_ref, lse_ref,
                 w2_buf, w2t_buf, m_ref, s_ref, sems, sem_t):
    i = pl.program_id(0)

    @pl.when(i == 0)
    def _():
        m_ref[0] = -jnp.inf
        s_ref[0] = 0.0
        for b in range(_K - 1):
            pltpu.make_async_copy(
                w2_hbm.at[:, pl.ds(b * _NT, _NT)],
                w2_buf.at[b], sems.at[b]).start()
        pltpu.make_async_copy(
            w2_hbm.at[:, pl.ds(_NFULL * _NT, _TAIL)],
            w2t_buf, sem_t).start()

    @pl.when(i + _K - 1 < _NFULL)
    def _():
        blk = i + _K - 1
        pltpu.make_async_copy(
            w2_hbm.at[:, pl.ds(blk * _NT, _NT)],
            w2_buf.at[blk % _K], sems.at[blk % _K]).start()

    h = h_ref[...]
    m_old = m_ref[0]
    s_old = s_ref[0]

    @pl.when(i < _NFULL)
    def _():
        slot = i % _K
        pltpu.make_async_copy(
            w2_hbm.at[:, pl.ds(0, _NT)],
            w2_buf.at[slot], sems.at[slot]).wait()
        rows = [jnp.dot(h, w2_buf[slot, :, pl.ds(r * _NT8, _NT8)],
                        preferred_element_type=jnp.float32)
                for r in range(8)]
        o = jnp.concatenate(rows, axis=0) + b2_ref[0]
        o_ref[0] = o
        m_new = jnp.maximum(m_old, jnp.max(o))
        s_ref[0] = s_old * jnp.exp(m_old - m_new) + jnp.sum(
            jnp.exp(o - m_new))
        m_ref[0] = m_new

    @pl.when(i == _NFULL)
    def _():
        pltpu.make_async_copy(
            w2_hbm.at[:, pl.ds(_NFULL * _NT, _TAIL)],
            w2t_buf, sem_t).wait()
        o_t = jnp.dot(h, w2t_buf[...],
                      preferred_element_type=jnp.float32)
        o_t = o_t + b2_ref[0, 0:1, 0:_TAIL]
        o_ref[0, 0:1, 0:_TAIL] = o_t
        m_new = jnp.maximum(m_old, jnp.max(o_t))
        s_new = s_old * jnp.exp(m_old - m_new) + jnp.sum(
            jnp.exp(o_t - m_new))
        lse_ref[0, 0] = m_new + jnp.log(s_new)


def _sub_body(o_ref, lse_ref, out_ref):
    out_ref[...] = o_ref[...] - lse_ref[0, 0]


def kernel(x, emb, W1, b1, W2, b2):
    w1r = W1.reshape(_CTX, _DIM, _HID)

    h = pl.pallas_call(
        _embed_body,
        grid_spec=pltpu.PrefetchScalarGridSpec(
            num_scalar_prefetch=1,
            grid=(_CTX,),
            in_specs=[
                pl.BlockSpec((8, _DIM), lambda i, xs: (xs[i] // 8, 0)),
                pl.BlockSpec((1, _DIM, _HID), lambda i, xs: (i, 0, 0)),
                pl.BlockSpec((1, _HID), lambda i, xs: (0, 0)),
            ],
            out_specs=pl.BlockSpec((1, _HID), lambda i, xs: (0, 0)),
            scratch_shapes=[pltpu.VMEM((1, _HID), jnp.float32)],
        ),
        out_shape=jax.ShapeDtypeStruct((1, _HID), jnp.float32),
        compiler_params=pltpu.CompilerParams(
            dimension_semantics=("arbitrary",)),
    )(x.astype(jnp.int32), emb, w1r, b1.reshape(1, _HID))

    b22 = jnp.pad(b2, (0, _VPAD - _VOCAB)).reshape(_NBLK, 8, _NT8)

    o2, lse = pl.pallas_call(
        _stream_body,
        grid=(_NBLK,),
        in_specs=[
            pl.BlockSpec((1, _HID), lambda i: (0, 0)),
            pl.BlockSpec(memory_space=pltpu.MemorySpace.HBM),
            pl.BlockSpec((1, 8, _NT8), lambda i: (i, 0, 0)),
        ],
        out_specs=[
            pl.BlockSpec((1, 8, _NT8), lambda i: (i, 0, 0)),
            pl.BlockSpec(memory_space=pltpu.SMEM),
        ],
        out_shape=[
            jax.ShapeDtypeStruct((_NBLK, 8, _NT8), jnp.float32),
            jax.ShapeDtypeStruct((1, 1), jnp.float32),
        ],
        scratch_shapes=[
            pltpu.VMEM((_K, _HID, _NT), jnp.float32),
            pltpu.VMEM((_HID, _TAIL), jnp.float32),
            pltpu.SMEM((1,), jnp.float32),
            pltpu.SMEM((1,), jnp.float32),
            pltpu.SemaphoreType.DMA((_K,)),
            pltpu.SemaphoreType.DMA,
        ],
        compiler_params=pltpu.CompilerParams(
            dimension_semantics=("arbitrary",),
            vmem_limit_bytes=60 * 1024 * 1024),
    )(h, W2, b22)

    lp2 = pl.pallas_call(
        _sub_body,
        grid=(_NBLK,),
        in_specs=[
            pl.BlockSpec((1, 8, _NT8), lambda i: (i, 0, 0)),
            pl.BlockSpec(memory_space=pltpu.SMEM),
        ],
        out_specs=pl.BlockSpec((1, 8, _NT8), lambda i: (i, 0, 0)),
        out_shape=jax.ShapeDtypeStruct((_NBLK, 8, _NT8), jnp.float32),
    )(o2, lse)

    return lp2.reshape(1, _VPAD)[:, :_VOCAB]


# speedup vs baseline: 4.1713x; 4.1713x over previous
import jax
import jax.numpy as jnp
from jax import lax
from jax.experimental import pallas as pl
from jax.experimental.pallas import tpu as pltpu

_VOCAB = 1000000
_DIM = 64
_CTX = 20
_HID = 128
_NT = 8192
_NT8 = _NT // 8
_NFULL = _VOCAB // _NT
_TAIL = _VOCAB - _NFULL * _NT
_NBLK = _NFULL + 1
_VPAD = _NBLK * _NT
_K = 6

_RDIMS = (((1,), (1,)), ((), ()))


def _embed_body(x_ref, embt_blk, w1_ref, b1_ref, h_ref, acc_ref):
    i = pl.program_id(0)

    @pl.when(i == 0)
    def _():
        acc_ref[...] = jnp.zeros_like(acc_ref)

    lane = x_ref[i] % 128
    sel = lax.broadcasted_iota(jnp.int32, (_DIM, 128), 1) == lane
    e_col = jnp.sum(jnp.where(sel, embt_blk[...], 0.0), axis=1,
                    keepdims=True)
    acc_ref[...] += lax.dot_general(
        e_col, w1_ref[0], (((0,), (0,)), ((), ())),
        preferred_element_type=jnp.float32)

    @pl.when(i == _CTX - 1)
    def _():
        h_ref[...] = jnp.maximum(acc_ref[...] + b1_ref[...], 0.0)


def _stream_body(h_ref, w2t_hbm, b2_ref, o_ref, lse_ref,
                 w2_buf, w2t_buf, m_ref, s_ref, sems, sem_t):
    i = pl.program_id(0)

    @pl.when(i == 0)
    def _():
        m_ref[0] = -jnp.inf
        s_ref[0] = 0.0
        for b in range(_K - 1):
            pltpu.make_async_copy(
                w2t_hbm.at[pl.ds(b * _NT, _NT), :],
                w2_buf.at[b], sems.at[b]).start()
        pltpu.make_async_copy(
            w2t_hbm.at[pl.ds(_NFULL * _NT, _TAIL), :],
            w2t_buf, sem_t).start()

    @pl.when(i + _K - 1 < _NFULL)
    def _():
        blk = i + _K - 1
        pltpu.make_async_copy(
            w2t_hbm.at[pl.ds(blk * _NT, _NT), :],
            w2_buf.at[blk % _K], sems.at[blk % _K]).start()

    h = h_ref[...]
    m_old = m_ref[0]
    s_old = s_ref[0]

    @pl.when(i < _NFULL)
    def _():
        slot = i % _K
        pltpu.make_async_copy(
            w2t_hbm.at[pl.ds(0, _NT), :],
            w2_buf.at[slot], sems.at[slot]).wait()
        rows = [lax.dot_general(h, w2_buf[slot, pl.ds(r * _NT8, _NT8), :],
                                _RDIMS, preferred_element_type=jnp.float32)
                for r in range(8)]
        o = jnp.concatenate(rows, axis=0) + b2_ref[0]
        o_ref[0] = o
        m_new = jnp.maximum(m_old, jnp.max(o))
        s_ref[0] = s_old * jnp.exp(m_old - m_new) + jnp.sum(
            jnp.exp(o - m_new))
        m_ref[0] = m_new

    @pl.when(i == _NFULL)
    def _():
        pltpu.make_async_copy(
            w2t_hbm.at[pl.ds(_NFULL * _NT, _TAIL), :],
            w2t_buf, sem_t).wait()
        o_t = lax.dot_general(h, w2t_buf[...], _RDIMS,
                              preferred_element_type=jnp.float32)
        o_t = o_t + b2_ref[0, 0:1, 0:_TAIL]
        o_ref[0, 0:1, 0:_TAIL] = o_t
        m_new = jnp.maximum(m_old, jnp.max(o_t))
        s_new = s_old * jnp.exp(m_old - m_new) + jnp.sum(
            jnp.exp(o_t - m_new))
        lse_ref[0, 0] = m_new + jnp.log(s_new)


def _sub_body(o_ref, lse_ref, out_ref):
    out_ref[...] = o_ref[...] - lse_ref[0, 0]


def kernel(x, emb, W1, b1, W2, b2):
    w1r = W1.reshape(_CTX, _DIM, _HID)
    embt = emb.T
    w2t = W2.T

    h = pl.pallas_call(
        _embed_body,
        grid_spec=pltpu.PrefetchScalarGridSpec(
            num_scalar_prefetch=1,
            grid=(_CTX,),
            in_specs=[
                pl.BlockSpec((_DIM, 128), lambda i, xs: (0, xs[i] // 128)),
                pl.BlockSpec((1, _DIM, _HID), lambda i, xs: (i, 0, 0)),
                pl.BlockSpec((1, _HID), lambda i, xs: (0, 0)),
            ],
            out_specs=pl.BlockSpec((1, _HID), lambda i, xs: (0, 0)),
            scratch_shapes=[pltpu.VMEM((1, _HID), jnp.float32)],
        ),
        out_shape=jax.ShapeDtypeStruct((1, _HID), jnp.float32),
        compiler_params=pltpu.CompilerParams(
            dimension_semantics=("arbitrary",)),
    )(x.astype(jnp.int32), embt, w1r, b1.reshape(1, _HID))

    b22 = jnp.pad(b2, (0, _VPAD - _VOCAB)).reshape(_NBLK, 8, _NT8)

    o2, lse = pl.pallas_call(
        _stream_body,
        grid=(_NBLK,),
        in_specs=[
            pl.BlockSpec((1, _HID), lambda i: (0, 0)),
            pl.BlockSpec(memory_space=pltpu.MemorySpace.HBM),
            pl.BlockSpec((1, 8, _NT8), lambda i: (i, 0, 0)),
        ],
        out_specs=[
            pl.BlockSpec((1, 8, _NT8), lambda i: (i, 0, 0)),
            pl.BlockSpec(memory_space=pltpu.SMEM),
        ],
        out_shape=[
            jax.ShapeDtypeStruct((_NBLK, 8, _NT8), jnp.float32),
            jax.ShapeDtypeStruct((1, 1), jnp.float32),
        ],
        scratch_shapes=[
            pltpu.VMEM((_K, _NT, _HID), jnp.float32),
            pltpu.VMEM((_TAIL, _HID), jnp.float32),
            pltpu.SMEM((1,), jnp.float32),
            pltpu.SMEM((1,), jnp.float32),
            pltpu.SemaphoreType.DMA((_K,)),
            pltpu.SemaphoreType.DMA,
        ],
        compiler_params=pltpu.CompilerParams(
            dimension_semantics=("arbitrary",),
            vmem_limit_bytes=60 * 1024 * 1024),
    )(h, w2t, b22)

    lp2 = pl.pallas_call(
        _sub_body,
        grid=(_NBLK,),
        in_specs=[
            pl.BlockSpec((1, 8, _NT8), lambda i: (i, 0, 0)),
            pl.BlockSpec(memory_space=pltpu.SMEM),
        ],
        out_specs=pl.BlockSpec((1, 8, _NT8), lambda i: (i, 0, 0)),
        out_shape=jax.ShapeDtypeStruct((_NBLK, 8, _NT8), jnp.float32),
    )(o2, lse)

    return lp2.reshape(1, _VPAD)[:, :_VOCAB]


# speedup vs baseline: 5.4107x; 1.2971x over previous
import jax
import jax.numpy as jnp
from jax import lax
from jax.experimental import pallas as pl
from jax.experimental.pallas import tpu as pltpu

_VOCAB = 1000000
_DIM = 64
_CTX = 20
_HID = 128
_NT = 8192
_NT8 = _NT // 8
_NFULL = _VOCAB // _NT
_TAIL = _VOCAB - _NFULL * _NT
_NBLK = _NFULL + 1
_VPAD = _NBLK * _NT
_K = 6

_RDIMS = (((1,), (1,)), ((), ()))


def _embed_body(x_ref, embt_blk, w1_ref, b1_ref, h_ref, acc_ref):
    i = pl.program_id(0)

    @pl.when(i == 0)
    def _():
        acc_ref[...] = jnp.zeros_like(acc_ref)

    lane = x_ref[i] % 128
    sel = lax.broadcasted_iota(jnp.int32, (_DIM, 128), 1) == lane
    e_col = jnp.sum(jnp.where(sel, embt_blk[...], 0.0), axis=1,
                    keepdims=True)
    acc_ref[...] += lax.dot_general(
        e_col, w1_ref[0], (((0,), (0,)), ((), ())),
        preferred_element_type=jnp.float32)

    @pl.when(i == _CTX - 1)
    def _():
        h_ref[...] = jnp.maximum(acc_ref[...] + b1_ref[...], 0.0)


def _stream_body(h_ref, w2t_hbm, b2_ref, o_ref, lse_ref,
                 w2_buf, w2t_buf, m_ref, s_ref, sems, sem_t):
    i = pl.program_id(0)

    @pl.when(i == 0)
    def _():
        m_ref[0] = -jnp.inf
        s_ref[0] = 0.0
        for b in range(_K - 1):
            pltpu.make_async_copy(
                w2t_hbm.at[pl.ds(b * _NT, _NT), :],
                w2_buf.at[b], sems.at[b]).start()
        pltpu.make_async_copy(
            w2t_hbm.at[pl.ds(_NFULL * _NT, _TAIL), :],
            w2t_buf, sem_t).start()

    @pl.when(i + _K - 1 < _NFULL)
    def _():
        blk = i + _K - 1
        pltpu.make_async_copy(
            w2t_hbm.at[pl.ds(blk * _NT, _NT), :],
            w2_buf.at[blk % _K], sems.at[blk % _K]).start()

    h = h_ref[...]
    m_old = m_ref[0]
    s_old = s_ref[0]

    @pl.when(i < _NFULL)
    def _():
        slot = i % _K
        pltpu.make_async_copy(
            w2t_hbm.at[pl.ds(0, _NT), :],
            w2_buf.at[slot], sems.at[slot]).wait()
        rows = [lax.dot_general(h, w2_buf[slot, pl.ds(r * _NT8, _NT8), :],
                                _RDIMS, preferred_element_type=jnp.float32)
                for r in range(8)]
        o = jnp.concatenate(rows, axis=0) + b2_ref[0]
        o_ref[0] = o
        m_new = jnp.maximum(m_old, jnp.max(o))
        s_ref[0] = s_old * jnp.exp(m_old - m_new) + jnp.sum(
            jnp.exp(o - m_new))
        m_ref[0] = m_new

    @pl.when(i == _NFULL)
    def _():
        pltpu.make_async_copy(
            w2t_hbm.at[pl.ds(_NFULL * _NT, _TAIL), :],
            w2t_buf, sem_t).wait()
        o_t = lax.dot_general(h, w2t_buf[...], _RDIMS,
                              preferred_element_type=jnp.float32)
        o_t = o_t + b2_ref[0, 0:1, 0:_TAIL]
        o_ref[0, 0:1, 0:_TAIL] = o_t
        m_new = jnp.maximum(m_old, jnp.max(o_t))
        s_new = s_old * jnp.exp(m_old - m_new) + jnp.sum(
            jnp.exp(o_t - m_new))
        lse_ref[0, 0] = m_new + jnp.log(s_new)


def _sub_body(o_ref, lse_ref, out_ref):
    out_ref[...] = o_ref[...] - lse_ref[0, 0]


def kernel(x, emb, W1, b1, W2, b2):
    w1r = W1.reshape(_CTX, _DIM, _HID)
    embt = emb.T
    w2t = W2.T

    h = pl.pallas_call(
        _embed_body,
        grid_spec=pltpu.PrefetchScalarGridSpec(
            num_scalar_prefetch=1,
            grid=(_CTX,),
            in_specs=[
                pl.BlockSpec((_DIM, 128), lambda i, xs: (0, xs[i] // 128)),
                pl.BlockSpec((1, _DIM, _HID), lambda i, xs: (i, 0, 0)),
                pl.BlockSpec((1, _HID), lambda i, xs: (0, 0)),
            ],
            out_specs=pl.BlockSpec((1, _HID), lambda i, xs: (0, 0)),
            scratch_shapes=[pltpu.VMEM((1, _HID), jnp.float32)],
        ),
        out_shape=jax.ShapeDtypeStruct((1, _HID), jnp.float32),
        compiler_params=pltpu.CompilerParams(
            dimension_semantics=("arbitrary",)),
    )(x.astype(jnp.int32), embt, w1r, b1.reshape(1, _HID))

    b22 = jnp.pad(b2, (0, _VPAD - _VOCAB)).reshape(_NBLK, 8, _NT8)

    o2, lse = pl.pallas_call(
        _stream_body,
        grid=(_NBLK,),
        in_specs=[
            pl.BlockSpec((1, _HID), lambda i: (0, 0)),
            pl.BlockSpec(memory_space=pltpu.MemorySpace.HBM),
            pl.BlockSpec((1, 8, _NT8), lambda i: (i, 0, 0)),
        ],
        out_specs=[
            pl.BlockSpec((1, 8, _NT8), lambda i: (i, 0, 0)),
            pl.BlockSpec(memory_space=pltpu.SMEM),
        ],
        out_shape=[
            jax.ShapeDtypeStruct((_NBLK, 8, _NT8), jnp.float32),
            jax.ShapeDtypeStruct((1, 1), jnp.float32),
        ],
        scratch_shapes=[
            pltpu.VMEM((_K, _NT, _HID), jnp.float32),
            pltpu.VMEM((_TAIL, _HID), jnp.float32),
            pltpu.SMEM((1,), jnp.float32),
            pltpu.SMEM((1,), jnp.float32),
            pltpu.SemaphoreType.DMA((_K,)),
            pltpu.SemaphoreType.DMA,
        ],
        compiler_params=pltpu.CompilerParams(
            dimension_semantics=("arbitrary",),
            vmem_limit_bytes=60 * 1024 * 1024),
    )(h, w2t, b22)

    lp2 = pl.pallas_call(
        _sub_body,
        in_specs=[
            pl.BlockSpec(memory_space=pltpu.MemorySpace.VMEM),
            pl.BlockSpec(memory_space=pltpu.SMEM),
        ],
        out_specs=pl.BlockSpec(memory_space=pltpu.MemorySpace.VMEM),
        out_shape=jax.ShapeDtypeStruct((_NBLK, 8, _NT8), jnp.float32),
    )(o2, lse)

    return lp2.reshape(1, _VPAD)[:, :_VOCAB]


# speedup vs baseline: 5.5577x; 1.0272x over previous
import jax
import jax.numpy as jnp
from jax import lax
from jax.experimental import pallas as pl
from jax.experimental.pallas import tpu as pltpu

_VOCAB = 1000000
_DIM = 64
_CTX = 20
_HID = 128
_NT = 8192
_NT8 = _NT // 8
_NFULL = _VOCAB // _NT
_TAIL = _VOCAB - _NFULL * _NT
_NBLK = _NFULL + 1
_VPAD = _NBLK * _NT
_K = 6

_RDIMS = (((1,), (1,)), ((), ()))


def _embed_body(x_ref, embt_hbm, w1_ref, b1_ref, h_ref, ebuf, sem):
    for j in range(_CTX):
        base = (x_ref[j] // 128) * 128
        pltpu.make_async_copy(
            embt_hbm.at[:, pl.ds(base, 128)], ebuf.at[j], sem).start()

    lanes = lax.broadcasted_iota(jnp.int32, (_DIM, 128), 1)
    acc = jnp.zeros((1, _HID), jnp.float32)
    for j in range(_CTX):
        pltpu.make_async_copy(
            embt_hbm.at[:, pl.ds(0, 128)], ebuf.at[j], sem).wait()
        sel = lanes == x_ref[j] % 128
        e_col = jnp.sum(jnp.where(sel, ebuf[j], 0.0), axis=1,
                        keepdims=True)
        acc = acc + lax.dot_general(
            e_col, w1_ref[j], (((0,), (0,)), ((), ())),
            preferred_element_type=jnp.float32)
    h_ref[...] = jnp.maximum(acc + b1_ref[...], 0.0)


def _stream_body(h_ref, w2t_hbm, b2_ref, o_ref, lse_ref,
                 w2_buf, w2t_buf, m_ref, s_ref, sems, sem_t):
    i = pl.program_id(0)

    @pl.when(i == 0)
    def _():
        m_ref[0] = -jnp.inf
        s_ref[0] = 0.0
        for b in range(_K - 1):
            pltpu.make_async_copy(
                w2t_hbm.at[pl.ds(b * _NT, _NT), :],
                w2_buf.at[b], sems.at[b]).start()
        pltpu.make_async_copy(
            w2t_hbm.at[pl.ds(_NFULL * _NT, _TAIL), :],
            w2t_buf, sem_t).start()

    @pl.when(i + _K - 1 < _NFULL)
    def _():
        blk = i + _K - 1
        pltpu.make_async_copy(
            w2t_hbm.at[pl.ds(blk * _NT, _NT), :],
            w2_buf.at[blk % _K], sems.at[blk % _K]).start()

    h = h_ref[...]
    m_old = m_ref[0]
    s_old = s_ref[0]

    @pl.when(i < _NFULL)
    def _():
        slot = i % _K
        pltpu.make_async_copy(
            w2t_hbm.at[pl.ds(0, _NT), :],
            w2_buf.at[slot], sems.at[slot]).wait()
        rows = [lax.dot_general(h, w2_buf[slot, pl.ds(r * _NT8, _NT8), :],
                                _RDIMS, preferred_element_type=jnp.float32)
                for r in range(8)]
        o = jnp.concatenate(rows, axis=0) + b2_ref[0]
        o_ref[0] = o
        m_new = jnp.maximum(m_old, jnp.max(o))
        s_ref[0] = s_old * jnp.exp(m_old - m_new) + jnp.sum(
            jnp.exp(o - m_new))
        m_ref[0] = m_new

    @pl.when(i == _NFULL)
    def _():
        pltpu.make_async_copy(
            w2t_hbm.at[pl.ds(_NFULL * _NT, _TAIL), :],
            w2t_buf, sem_t).wait()
        o_t = lax.dot_general(h, w2t_buf[...], _RDIMS,
                              preferred_element_type=jnp.float32)
        o_t = o_t + b2_ref[0, 0:1, 0:_TAIL]
        o_ref[0, 0:1, 0:_TAIL] = o_t
        m_new = jnp.maximum(m_old, jnp.max(o_t))
        s_new = s_old * jnp.exp(m_old - m_new) + jnp.sum(
            jnp.exp(o_t - m_new))
        lse_ref[0, 0] = m_new + jnp.log(s_new)


def _sub_body(o_ref, lse_ref, out_ref):
    out_ref[...] = o_ref[...] - lse_ref[0, 0]


def kernel(x, emb, W1, b1, W2, b2):
    w1r = W1.reshape(_CTX, _DIM, _HID)
    embt = emb.T
    w2t = W2.T

    h = pl.pallas_call(
        _embed_body,
        in_specs=[
            pl.BlockSpec(memory_space=pltpu.MemorySpace.SMEM),
            pl.BlockSpec(memory_space=pltpu.MemorySpace.HBM),
            pl.BlockSpec(memory_space=pltpu.MemorySpace.VMEM),
            pl.BlockSpec(memory_space=pltpu.MemorySpace.VMEM),
        ],
        out_specs=pl.BlockSpec(memory_space=pltpu.MemorySpace.VMEM),
        out_shape=jax.ShapeDtypeStruct((1, _HID), jnp.float32),
        scratch_shapes=[
            pltpu.VMEM((_CTX, _DIM, 128), jnp.float32),
            pltpu.SemaphoreType.DMA,
        ],
    )(x.astype(jnp.int32), embt, w1r, b1.reshape(1, _HID))

    b22 = jnp.pad(b2, (0, _VPAD - _VOCAB)).reshape(_NBLK, 8, _NT8)

    o2, lse = pl.pallas_call(
        _stream_body,
        grid=(_NBLK,),
        in_specs=[
            pl.BlockSpec((1, _HID), lambda i: (0, 0)),
            pl.BlockSpec(memory_space=pltpu.MemorySpace.HBM),
            pl.BlockSpec((1, 8, _NT8), lambda i: (i, 0, 0)),
        ],
        out_specs=[
            pl.BlockSpec((1, 8, _NT8), lambda i: (i, 0, 0)),
            pl.BlockSpec(memory_space=pltpu.SMEM),
        ],
        out_shape=[
            jax.ShapeDtypeStruct((_NBLK, 8, _NT8), jnp.float32),
            jax.ShapeDtypeStruct((1, 1), jnp.float32),
        ],
        scratch_shapes=[
            pltpu.VMEM((_K, _NT, _HID), jnp.float32),
            pltpu.VMEM((_TAIL, _HID), jnp.float32),
            pltpu.SMEM((1,), jnp.float32),
            pltpu.SMEM((1,), jnp.float32),
            pltpu.SemaphoreType.DMA((_K,)),
            pltpu.SemaphoreType.DMA,
        ],
        compiler_params=pltpu.CompilerParams(
            dimension_semantics=("arbitrary",),
            vmem_limit_bytes=60 * 1024 * 1024),
    )(h, w2t, b22)

    lp2 = pl.pallas_call(
        _sub_body,
        in_specs=[
            pl.BlockSpec(memory_space=pltpu.MemorySpace.VMEM),
            pl.BlockSpec(memory_space=pltpu.SMEM),
        ],
        out_specs=pl.BlockSpec(memory_space=pltpu.MemorySpace.VMEM),
        out_shape=jax.ShapeDtypeStruct((_NBLK, 8, _NT8), jnp.float32),
    )(o2, lse)

    return lp2.reshape(1, _VPAD)[:, :_VOCAB]
